# Initial kernel scaffold; baseline (speedup 1.0000x reference)
import jax
import jax.numpy as jnp
from jax.experimental import pallas as pl


def kernel(x, emb_table, pos_table):
    raise NotImplementedError("write your pallas kernel here")


# speedup vs baseline: 1.4549x; 1.4549x over previous
import jax
import jax.numpy as jnp
from jax import lax
from jax.experimental import pallas as pl
from jax.experimental.pallas import tpu as pltpu
from jax.experimental.pallas import tpu_sc as plsc

SEQ = 2048
BATCH = 16
D = 64
L = 16
NC = 2
NS = 16
ROWS_PER_TILE = SEQ // NS
TOK_PER_TILE = ROWS_PER_TILE * BATCH
TOK_PER_WORKER = TOK_PER_TILE // NC
G = 128
CHUNK = 512
SCALE = 8.0


def _emb_body(x_hbm, emb_hbm, pos_hbm, out_hbm,
              x_v, pos_v, sum_v, all_v, shared, emb_rows, pos_rows, sem):
    cid = lax.axis_index("c")
    sid = lax.axis_index("s")
    tok0 = sid * TOK_PER_TILE

    pltpu.sync_copy(x_hbm.at[pl.ds(tok0, TOK_PER_TILE)], x_v)

    def sum_body(i, acc):
        row = x_v[pl.ds(i * L, L)]
        return acc + jnp.minimum(row, 1)

    tot = lax.fori_loop(0, ROWS_PER_TILE, sum_body, jnp.zeros((L,), jnp.int32))
    sum_v[...] = tot
    pltpu.sync_copy(sum_v, shared.at[pl.ds(sid * L, L)])
    plsc.subcore_barrier()
    pltpu.sync_copy(shared, all_v)

    def pre_body(w, acc):
        row = all_v[pl.ds(w * L, L)]
        flag = jnp.minimum(jnp.maximum(sid - w, 0), 1)
        return acc + row * flag

    prefix = lax.fori_loop(0, NS, pre_body, jnp.zeros((L,), jnp.int32))

    def pos_body(i, acc):
        row = x_v[pl.ds(i * L, L)]
        m = jnp.minimum(row, 1)
        acc = acc + m
        pos_v[pl.ds(i * L, L)] = acc * m
        return acc

    lax.fori_loop(0, ROWS_PER_TILE, pos_body, prefix)

    off = cid * TOK_PER_WORKER
    out0 = tok0 + off
    for h in range(TOK_PER_WORKER // CHUNK):
        base = off + h * CHUNK
        copies = []
        for j in range(CHUNK // G):
            copies.append(pltpu.async_copy(
                emb_hbm.at[x_v.at[pl.ds(base + j * G, G)]],
                emb_rows.at[pl.ds(j * G, G)], sem))
            copies.append(pltpu.async_copy(
                pos_hbm.at[pos_v.at[pl.ds(base + j * G, G)]],
                pos_rows.at[pl.ds(j * G, G)], sem))
        for cpy in copies:
            cpy.wait()

        def fma_body(r, _):
            for k in range(D // L):
                e = emb_rows[r, pl.ds(k * L, L)]
                p = pos_rows[r, pl.ds(k * L, L)]
                emb_rows[r, pl.ds(k * L, L)] = e * SCALE + p
            return 0

        lax.fori_loop(0, CHUNK, fma_body, 0)
        pltpu.sync_copy(emb_rows, out_hbm.at[pl.ds(out0 + h * CHUNK, CHUNK)])


def kernel(x, emb_table, pos_table):
    x_flat = x.reshape(SEQ * BATCH)
    mesh = plsc.VectorSubcoreMesh(core_axis_name="c", subcore_axis_name="s")
    out = pl.kernel(
        _emb_body,
        out_type=jax.ShapeDtypeStruct((SEQ * BATCH, D), jnp.float32),
        mesh=mesh,
        compiler_params=pltpu.CompilerParams(use_tc_tiling_on_sc=False),
        scratch_types=[
            pltpu.VMEM((TOK_PER_TILE,), jnp.int32),
            pltpu.VMEM((TOK_PER_TILE,), jnp.int32),
            pltpu.VMEM((L,), jnp.int32),
            pltpu.VMEM((NS * L,), jnp.int32),
            pltpu.VMEM_SHARED((NS * L,), jnp.int32),
            pltpu.VMEM((CHUNK, D), jnp.float32),
            pltpu.VMEM((CHUNK, D), jnp.float32),
            pltpu.SemaphoreType.DMA,
        ],
    )(x_flat, emb_table, pos_table)
    return out.reshape(SEQ, BATCH, D)


# speedup vs baseline: 2.2948x; 1.5773x over previous
import jax
import jax.numpy as jnp
from jax import lax
from jax.experimental import pallas as pl
from jax.experimental.pallas import tpu as pltpu
from jax.experimental.pallas import tpu_sc as plsc

SEQ = 2048
BATCH = 16
D = 64
V = 100000
PV = 2049
L = 16
NC = 2
NS = 16
NW = NC * NS
SCALE = 8.0
VECS = SEQ // L


def _emb_body(x_hbm, emb_hbm, pos_hbm, out_hbm,
              xcol_v, poscol_v, row_v, prow_v, idx_v, pidx_v, acc_v,
              xbuf_sh, posbuf_sh, sem):
    cid = lax.axis_index("c")
    sid = lax.axis_index("s")
    wid = cid * NS + sid

    pltpu.sync_copy(x_hbm.at[sid], xcol_v)

    def scan_body(k, carry):
        v = xcol_v[pl.ds(k * L, L)]
        m = jnp.minimum(v, 1)
        cs = plsc.cumsum(m)
        poscol_v[pl.ds(k * L, L)] = (cs + carry) * m
        return carry + jnp.max(cs)

    lax.fori_loop(0, VECS, scan_body, jnp.int32(0))
    pltpu.sync_copy(xcol_v, xbuf_sh.at[pl.ds(sid * SEQ, SEQ)])
    pltpu.sync_copy(poscol_v, posbuf_sh.at[pl.ds(sid * SEQ, SEQ)])
    plsc.subcore_barrier()

    for r in range(D // NW):
        d = wid + r * NW
        cp_row = pltpu.async_copy(emb_hbm.at[d], row_v, sem)
        cp_prow = pltpu.async_copy(pos_hbm.at[d], prow_v, sem)
        cp_row.wait()
        cp_prow.wait()
        for b in range(BATCH):
            pltpu.sync_copy(xbuf_sh.at[pl.ds(b * SEQ, SEQ)], idx_v)
            pltpu.sync_copy(posbuf_sh.at[pl.ds(b * SEQ, SEQ)], pidx_v)

            def gat_body(k, _, base=0):
                for u in range(4):
                    o = (k * 4 + u) * L
                    tok = idx_v[pl.ds(o, L)]
                    pos = pidx_v[pl.ds(o, L)]
                    e = plsc.load_gather(row_v, [tok])
                    p = plsc.load_gather(prow_v, [pos])
                    acc_v[pl.ds(o, L)] = e * SCALE + p
                return 0

            lax.fori_loop(0, VECS // 4, gat_body, 0)
            pltpu.sync_copy(acc_v, out_hbm.at[b, d])


def kernel(x, emb_table, pos_table):
    x_t = x.T
    emb_t = emb_table.T
    pos_t = pos_table.T
    mesh = plsc.VectorSubcoreMesh(core_axis_name="c", subcore_axis_name="s")
    out_t = pl.kernel(
        _emb_body,
        out_type=jax.ShapeDtypeStruct((BATCH, D, SEQ), jnp.float32),
        mesh=mesh,
        compiler_params=pltpu.CompilerParams(
            use_tc_tiling_on_sc=True, needs_layout_passes=False),
        scratch_types=[
            pltpu.VMEM((SEQ,), jnp.int32),
            pltpu.VMEM((SEQ,), jnp.int32),
            pltpu.VMEM((V,), jnp.float32),
            pltpu.VMEM((PV,), jnp.float32),
            pltpu.VMEM((SEQ,), jnp.int32),
            pltpu.VMEM((SEQ,), jnp.int32),
            pltpu.VMEM((SEQ,), jnp.float32),
            pltpu.VMEM_SHARED((BATCH * SEQ,), jnp.int32),
            pltpu.VMEM_SHARED((BATCH * SEQ,), jnp.int32),
            pltpu.SemaphoreType.DMA,
        ],
    )(x_t, emb_t, pos_t)
    return out_t.transpose(2, 0, 1)


# speedup vs baseline: 2.4854x; 1.0830x over previous
import jax
import jax.numpy as jnp
from jax import lax
from jax.experimental import pallas as pl
from jax.experimental.pallas import tpu as pltpu
from jax.experimental.pallas import tpu_sc as plsc

SEQ = 2048
BATCH = 16
D = 64
V = 100000
PV = 2049
L = 16
NC = 2
NS = 16
NW = NC * NS
SCALE = 8.0
VECS = SEQ // L
UNROLL = 4


def _emb_body(x_hbm, emb_hbm, pos_hbm, out_hbm,
              xcol_v, poscol_v, row_v, prow_v,
              idx0, pidx0, idx1, pidx1, acc0, acc1,
              xbuf_hbm, posbuf_hbm, sem_row, sem_in, sem_out):
    cid = lax.axis_index("c")
    sid = lax.axis_index("s")
    wid = cid * NS + sid

    cp_row = pltpu.async_copy(emb_hbm.at[wid], row_v, sem_row)
    cp_prow = pltpu.async_copy(pos_hbm.at[wid], prow_v, sem_row)

    pltpu.sync_copy(x_hbm.at[sid], xcol_v)

    def scan_body(k, carry):
        v = xcol_v[pl.ds(k * L, L)]
        m = jnp.minimum(v, 1)
        cs = plsc.cumsum(m)
        poscol_v[pl.ds(k * L, L)] = (cs + carry) * m
        return carry + jnp.max(cs)

    lax.fori_loop(0, VECS, scan_body, jnp.int32(0))
    pltpu.sync_copy(xcol_v, xbuf_hbm.at[pl.ds(sid * SEQ, SEQ)])
    pltpu.sync_copy(poscol_v, posbuf_hbm.at[pl.ds(sid * SEQ, SEQ)])
    plsc.subcore_barrier()

    ibufs = ((idx0, pidx0), (idx1, pidx1))
    abufs = (acc0, acc1)
    steps = [(r, b) for r in range(D // NW) for b in range(BATCH)]

    def prefetch(step, slot):
        _, b = step
        return (pltpu.async_copy(xbuf_hbm.at[pl.ds(b * SEQ, SEQ)],
                                 ibufs[slot][0], sem_in),
                pltpu.async_copy(posbuf_hbm.at[pl.ds(b * SEQ, SEQ)],
                                 ibufs[slot][1], sem_in))

    pf = {0: prefetch(steps[0], 0)}
    store_h = {}
    for i, (r, b) in enumerate(steps):
        p = i % 2
        d = wid + r * NW
        if b == 0:
            if r == 0:
                cp_row.wait()
                cp_prow.wait()
            else:
                pltpu.sync_copy(emb_hbm.at[d], row_v)
                pltpu.sync_copy(pos_hbm.at[d], prow_v)
        for h in pf.pop(i):
            h.wait()
        if i + 1 < len(steps):
            pf[i + 1] = prefetch(steps[i + 1], 1 - p)
        if p in store_h:
            store_h.pop(p).wait()
        idx_v, pidx_v = ibufs[p]
        acc_v = abufs[p]

        def gat_body(k, _):
            for u in range(UNROLL):
                o = (k * UNROLL + u) * L
                tok = idx_v[pl.ds(o, L)]
                pos = pidx_v[pl.ds(o, L)]
                e = plsc.load_gather(row_v, [tok])
                pe = plsc.load_gather(prow_v, [pos])
                acc_v[pl.ds(o, L)] = e * SCALE + pe
            return 0

        lax.fori_loop(0, VECS // UNROLL, gat_body, 0)
        store_h[p] = pltpu.async_copy(acc_v, out_hbm.at[b, d], sem_out)
    for h in store_h.values():
        h.wait()


def kernel(x, emb_table, pos_table):
    x_t = x.T
    emb_t = emb_table.T
    pos_t = pos_table.T
    mesh = plsc.VectorSubcoreMesh(core_axis_name="c", subcore_axis_name="s")
    out_t = pl.kernel(
        _emb_body,
        out_type=jax.ShapeDtypeStruct((BATCH, D, SEQ), jnp.float32),
        mesh=mesh,
        compiler_params=pltpu.CompilerParams(
            use_tc_tiling_on_sc=True, needs_layout_passes=False),
        scratch_types=[
            pltpu.VMEM((SEQ,), jnp.int32),
            pltpu.VMEM((SEQ,), jnp.int32),
            pltpu.VMEM((V,), jnp.float32),
            pltpu.VMEM((PV,), jnp.float32),
            pltpu.VMEM((SEQ,), jnp.int32),
            pltpu.VMEM((SEQ,), jnp.int32),
            pltpu.VMEM((SEQ,), jnp.int32),
            pltpu.VMEM((SEQ,), jnp.int32),
            pltpu.VMEM((SEQ,), jnp.float32),
            pltpu.VMEM((SEQ,), jnp.float32),
            pltpu.HBM((BATCH * SEQ,), jnp.int32),
            pltpu.HBM((BATCH * SEQ,), jnp.int32),
            pltpu.SemaphoreType.DMA,
            pltpu.SemaphoreType.DMA,
            pltpu.SemaphoreType.DMA,
        ],
    )(x_t, emb_t, pos_t)
    return out_t.transpose(2, 0, 1)
